# baseline (speedup 1.0000x reference)
import functools

import jax
import jax.numpy as jnp
from jax import lax
from jax.experimental import pallas as pl
from jax.experimental.pallas import tpu as pltpu
from jax.experimental.pallas import tpu_sc as plsc

R = 2 ** 18
F = 48
N = 1000000
C = 160
NCH = N // C
NW = 32
SEG = F // 16
NG = C // 16


def _body(px_hbm, py_hbm, cb_hbm, out_hbm,
          px_v, py_v, idx_a, idx_b, w_a, w_b, feat_a, feat_b, trans_v,
          sem_a, sem_b):
    wid = lax.axis_index("s") * 2 + lax.axis_index("c")
    nch_w = (NCH - wid + NW - 1) // NW
    lanes = jnp.arange(16, dtype=jnp.int32)
    lim = jnp.float32(R - 1 - 1e-5)
    top = jnp.full((16,), R - 1, jnp.int32)

    def stage_idx(cid, idx_v, w_v):
        base = cid * C
        pltpu.sync_copy(px_hbm.at[pl.ds(base, C)], px_v)
        pltpu.sync_copy(py_hbm.at[pl.ds(base, C)], py_v)

        def idx_body(i, carry):
            sl = pl.ds(i * 16, 16)
            x = jnp.maximum(jnp.minimum(px_v[sl] * (R - 1.0), lim), 0.0)
            y = jnp.maximum(jnp.minimum(py_v[sl] * (R - 1.0), lim), 0.0)
            xi = x.astype(jnp.int32)
            yi = y.astype(jnp.int32)
            x2 = jnp.minimum(xi + 1, top)
            y2 = jnp.minimum(yi + 1, top)
            idx_v[sl] = xi * 2
            idx_v[pl.ds(C + i * 16, 16)] = x2 * 2
            idx_v[pl.ds(2 * C + i * 16, 16)] = yi * 2 + 1
            idx_v[pl.ds(3 * C + i * 16, 16)] = y2 * 2 + 1
            w_v[sl] = x2.astype(jnp.float32) - x
            w_v[pl.ds(C + i * 16, 16)] = x - xi.astype(jnp.float32)
            w_v[pl.ds(2 * C + i * 16, 16)] = y2.astype(jnp.float32) - y
            w_v[pl.ds(3 * C + i * 16, 16)] = y - yi.astype(jnp.float32)
            return carry

        lax.fori_loop(0, NG, idx_body, 0)

    fvecs = [lanes + 16 * s for s in range(SEG)]

    def stage_combine_out(cid, w_v, feat_v):
        def comb_body(g, carry):
            w1g = w_v[pl.ds(g * 16, 16)]
            w2g = w_v[pl.ds(C + g * 16, 16)]
            w3g = w_v[pl.ds(2 * C + g * 16, 16)]
            w4g = w_v[pl.ds(3 * C + g * 16, 16)]
            for l in range(16):
                c = g * 16 + l
                w1 = w1g[l]
                w2 = w2g[l]
                w3 = w3g[l]
                w4 = w4g[l]
                cs = jnp.full((16,), c, jnp.int32)
                for s in range(SEG):
                    sl = pl.ds(s * 16, 16)
                    a = feat_v[c, sl]
                    b = feat_v[C + c, sl]
                    cc = feat_v[2 * C + c, sl]
                    d = feat_v[3 * C + c, sl]
                    r = (w1 * a + w2 * b) + (w3 * cc + w4 * d)
                    plsc.store_scatter(trans_v, [fvecs[s], cs], r)
            return carry

        lax.fori_loop(0, NG, comb_body, 0)
        pltpu.sync_copy(trans_v.at[:, pl.ds(0, C)],
                        out_hbm.at[:, pl.ds(cid * C, C)])

    def gather_start(idx_v, feat_v, sem):
        return pltpu.async_copy(cb_hbm.at[idx_v], feat_v, sem)

    stage_idx(wid, idx_a, w_a)
    gather_start(idx_a, feat_a, sem_a)

    npairs = (nch_w + 1) // 2

    def pair_body(j, carry):
        ka = 2 * j
        kb = 2 * j + 1
        cid_a = wid + ka * NW
        cid_b = wid + kb * NW

        @pl.when(kb < nch_w)
        def _():
            stage_idx(cid_b, idx_b, w_b)
            gather_start(idx_b, feat_b, sem_b)

        pltpu.make_async_copy(cb_hbm.at[idx_a], feat_a, sem_a).wait()
        stage_combine_out(cid_a, w_a, feat_a)

        @pl.when(kb < nch_w)
        def _():
            @pl.when(kb + 1 < nch_w)
            def _():
                stage_idx(cid_b + NW, idx_a, w_a)
                gather_start(idx_a, feat_a, sem_a)

            pltpu.make_async_copy(cb_hbm.at[idx_b], feat_b, sem_b).wait()
            stage_combine_out(cid_b, w_b, feat_b)

        return carry

    lax.fori_loop(0, npairs, pair_body, 0)


CC = 256
CPW = R // NW


def _tbody(cbt_hbm, out_hbm, in_a, in_b, out_v, sem_a, sem_b):
    wid = lax.axis_index("s") * 2 + lax.axis_index("c")
    col0 = wid * CPW
    nch = CPW // CC

    def fetch(k, buf, sem):
        return pltpu.async_copy(
            cbt_hbm.at[:, pl.ds(col0 + k * CC, CC)], buf, sem)

    lanes = jnp.arange(16, dtype=jnp.int32)
    rots = [jnp.bitwise_and(lanes + k, 15) for k in range(16)]
    outv = [r * 96 + lanes for r in rots]

    def transpose_out(k, in_v):
        def cb_body(cb, carry):
            c0 = cb * 16
            for f0 in range(0, 96, 16):
                fi = lanes + f0
                for kk in range(16):
                    v = plsc.load_gather(in_v, [fi, c0 + rots[kk]])
                    plsc.store_scatter(out_v, [(c0 * 96 + f0) + outv[kk]], v)
            return carry

        lax.fori_loop(0, CC // 16, cb_body, 0)
        pltpu.sync_copy(out_v,
                        out_hbm.at[pl.ds((col0 + k * CC) * 96, CC * 96)])

    fetch(0, in_a, sem_a)

    def pair_body(j, carry):
        ka = 2 * j
        kb = 2 * j + 1
        pltpu.make_async_copy(
            cbt_hbm.at[:, pl.ds(col0 + ka * CC, CC)], in_a, sem_a).wait()
        fetch(kb, in_b, sem_b)
        transpose_out(ka, in_a)

        @pl.when(kb + 1 < nch)
        def _():
            fetch(kb + 1, in_a, sem_a)

        pltpu.make_async_copy(
            cbt_hbm.at[:, pl.ds(col0 + kb * CC, CC)], in_b, sem_b).wait()
        transpose_out(kb, in_b)
        return carry

    lax.fori_loop(0, nch // 2, pair_body, 0)


def _transpose_table(cbt):
    run = functools.partial(
        pl.kernel,
        mesh=plsc.VectorSubcoreMesh(core_axis_name="c", subcore_axis_name="s"),
        out_type=jax.ShapeDtypeStruct((2 * R * F,), jnp.float32),
        scratch_types=[
            pltpu.VMEM((96, CC), jnp.float32),
            pltpu.VMEM((96, CC), jnp.float32),
            pltpu.VMEM((96 * CC,), jnp.float32),
            pltpu.SemaphoreType.DMA,
            pltpu.SemaphoreType.DMA,
        ],
        compiler_params=pltpu.CompilerParams(
            needs_layout_passes=False, use_tc_tiling_on_sc=True
        ),
    )(_tbody)
    return run(cbt)


@jax.jit
def kernel(pts, codebook_0):
    cb2 = _transpose_table(codebook_0.T).reshape(2 * R, F)
    px = pts[:, 0]
    py = pts[:, 1]
    run2 = functools.partial(
        pl.kernel,
        mesh=plsc.VectorSubcoreMesh(core_axis_name="c", subcore_axis_name="s"),
        out_type=jax.ShapeDtypeStruct((F, N), jnp.float32),
        scratch_types=[
            pltpu.VMEM((C,), jnp.float32),
            pltpu.VMEM((C,), jnp.float32),
            pltpu.VMEM((4 * C,), jnp.int32),
            pltpu.VMEM((4 * C,), jnp.int32),
            pltpu.VMEM((4 * C,), jnp.float32),
            pltpu.VMEM((4 * C,), jnp.float32),
            pltpu.VMEM((4 * C, F), jnp.float32),
            pltpu.VMEM((4 * C, F), jnp.float32),
            pltpu.VMEM((F, C + 1), jnp.float32),
            pltpu.SemaphoreType.DMA,
            pltpu.SemaphoreType.DMA,
        ],
        compiler_params=pltpu.CompilerParams(
            needs_layout_passes=False, use_tc_tiling_on_sc=False
        ),
    )(_body)
    return run2(px, py, cb2).T


# speedup vs baseline: 4.8255x; 4.8255x over previous
import functools

import jax
import jax.numpy as jnp
from jax import lax
from jax.experimental import pallas as pl
from jax.experimental.pallas import tpu as pltpu
from jax.experimental.pallas import tpu_sc as plsc

R = 2 ** 18
F = 48
N = 1000000
C = 320
NCH = N // C
NW = 32
SEG = F // 16
NG = C // 16


def _body(px_hbm, py_hbm, cb_hbm, out_hbm,
          px_v, py_v, idx_a, idx_b, w_a, w_b, feat_a, feat_b,
          sem_a, sem_b):
    wid = lax.axis_index("s") * 2 + lax.axis_index("c")
    nch_w = (NCH - wid + NW - 1) // NW
    lanes = jnp.arange(16, dtype=jnp.int32)
    lim = jnp.float32(R - 1 - 1e-5)
    top = jnp.full((16,), R - 1, jnp.int32)

    def stage_idx(cid, idx_v, w_v):
        base = cid * C
        pltpu.sync_copy(px_hbm.at[pl.ds(base, C)], px_v)
        pltpu.sync_copy(py_hbm.at[pl.ds(base, C)], py_v)

        def idx_body(i, carry):
            sl = pl.ds(i * 16, 16)
            x = jnp.maximum(jnp.minimum(px_v[sl] * (R - 1.0), lim), 0.0)
            y = jnp.maximum(jnp.minimum(py_v[sl] * (R - 1.0), lim), 0.0)
            xi = x.astype(jnp.int32)
            yi = y.astype(jnp.int32)
            x2 = jnp.minimum(xi + 1, top)
            y2 = jnp.minimum(yi + 1, top)
            idx_v[sl] = xi * 2
            idx_v[pl.ds(C + i * 16, 16)] = x2 * 2
            idx_v[pl.ds(2 * C + i * 16, 16)] = yi * 2 + 1
            idx_v[pl.ds(3 * C + i * 16, 16)] = y2 * 2 + 1
            w_v[sl] = x2.astype(jnp.float32) - x
            w_v[pl.ds(C + i * 16, 16)] = x - xi.astype(jnp.float32)
            w_v[pl.ds(2 * C + i * 16, 16)] = y2.astype(jnp.float32) - y
            w_v[pl.ds(3 * C + i * 16, 16)] = y - yi.astype(jnp.float32)
            return carry

        lax.fori_loop(0, NG, idx_body, 0)

    def stage_combine_out(cid, w_v, feat_v):
        def comb_body(g, carry):
            w1g = w_v[pl.ds(g * 16, 16)]
            w2g = w_v[pl.ds(C + g * 16, 16)]
            w3g = w_v[pl.ds(2 * C + g * 16, 16)]
            w4g = w_v[pl.ds(3 * C + g * 16, 16)]
            for l in range(16):
                c = g * 16 + l
                w1 = w1g[l]
                w2 = w2g[l]
                w3 = w3g[l]
                w4 = w4g[l]
                for s in range(SEG):
                    sl = pl.ds(s * 16, 16)
                    a = feat_v[c, sl]
                    b = feat_v[C + c, sl]
                    cc = feat_v[2 * C + c, sl]
                    d = feat_v[3 * C + c, sl]
                    feat_v[c, sl] = (w1 * a + w2 * b) + (w3 * cc + w4 * d)
            return carry

        lax.fori_loop(0, NG, comb_body, 0)
        pltpu.sync_copy(feat_v.at[pl.ds(0, C)],
                        out_hbm.at[pl.ds(cid * C, C)])

    def gather_start(idx_v, feat_v, sem):
        return pltpu.async_copy(cb_hbm.at[idx_v], feat_v, sem)

    stage_idx(wid, idx_a, w_a)
    gather_start(idx_a, feat_a, sem_a)

    npairs = (nch_w + 1) // 2

    def pair_body(j, carry):
        ka = 2 * j
        kb = 2 * j + 1
        cid_a = wid + ka * NW
        cid_b = wid + kb * NW

        @pl.when(kb < nch_w)
        def _():
            stage_idx(cid_b, idx_b, w_b)
            gather_start(idx_b, feat_b, sem_b)

        pltpu.make_async_copy(cb_hbm.at[idx_a], feat_a, sem_a).wait()
        stage_combine_out(cid_a, w_a, feat_a)

        @pl.when(kb < nch_w)
        def _():
            @pl.when(kb + 1 < nch_w)
            def _():
                stage_idx(cid_b + NW, idx_a, w_a)
                gather_start(idx_a, feat_a, sem_a)

            pltpu.make_async_copy(cb_hbm.at[idx_b], feat_b, sem_b).wait()
            stage_combine_out(cid_b, w_b, feat_b)

        return carry

    lax.fori_loop(0, npairs, pair_body, 0)


CC = 256
CPW = R // NW


def _tbody(cbt_hbm, out_hbm, in_a, in_b, out_v, sem_a, sem_b):
    wid = lax.axis_index("s") * 2 + lax.axis_index("c")
    col0 = wid * CPW
    nch = CPW // CC

    def fetch(k, buf, sem):
        return pltpu.async_copy(
            cbt_hbm.at[:, pl.ds(col0 + k * CC, CC)], buf, sem)

    lanes = jnp.arange(16, dtype=jnp.int32)
    rots = [jnp.bitwise_and(lanes + k, 15) for k in range(16)]
    outv = [r * 96 + lanes for r in rots]

    def transpose_out(k, in_v):
        def cb_body(cb, carry):
            c0 = cb * 16
            for f0 in range(0, 96, 16):
                fi = lanes + f0
                for kk in range(16):
                    v = plsc.load_gather(in_v, [fi, c0 + rots[kk]])
                    plsc.store_scatter(out_v, [(c0 * 96 + f0) + outv[kk]], v)
            return carry

        lax.fori_loop(0, CC // 16, cb_body, 0)
        pltpu.sync_copy(out_v,
                        out_hbm.at[pl.ds((col0 + k * CC) * 96, CC * 96)])

    fetch(0, in_a, sem_a)

    def pair_body(j, carry):
        ka = 2 * j
        kb = 2 * j + 1
        pltpu.make_async_copy(
            cbt_hbm.at[:, pl.ds(col0 + ka * CC, CC)], in_a, sem_a).wait()
        fetch(kb, in_b, sem_b)
        transpose_out(ka, in_a)

        @pl.when(kb + 1 < nch)
        def _():
            fetch(kb + 1, in_a, sem_a)

        pltpu.make_async_copy(
            cbt_hbm.at[:, pl.ds(col0 + kb * CC, CC)], in_b, sem_b).wait()
        transpose_out(kb, in_b)
        return carry

    lax.fori_loop(0, nch // 2, pair_body, 0)


def _transpose_table(cbt):
    run = functools.partial(
        pl.kernel,
        mesh=plsc.VectorSubcoreMesh(core_axis_name="c", subcore_axis_name="s"),
        out_type=jax.ShapeDtypeStruct((2 * R * F,), jnp.float32),
        scratch_types=[
            pltpu.VMEM((96, CC), jnp.float32),
            pltpu.VMEM((96, CC), jnp.float32),
            pltpu.VMEM((96 * CC,), jnp.float32),
            pltpu.SemaphoreType.DMA,
            pltpu.SemaphoreType.DMA,
        ],
        compiler_params=pltpu.CompilerParams(
            needs_layout_passes=False, use_tc_tiling_on_sc=True
        ),
    )(_tbody)
    return run(cbt)


@jax.jit
def kernel(pts, codebook_0):
    cb2 = _transpose_table(codebook_0.T).reshape(2 * R, F)
    px = pts[:, 0]
    py = pts[:, 1]
    run2 = functools.partial(
        pl.kernel,
        mesh=plsc.VectorSubcoreMesh(core_axis_name="c", subcore_axis_name="s"),
        out_type=jax.ShapeDtypeStruct((N, F), jnp.float32),
        scratch_types=[
            pltpu.VMEM((C,), jnp.float32),
            pltpu.VMEM((C,), jnp.float32),
            pltpu.VMEM((4 * C,), jnp.int32),
            pltpu.VMEM((4 * C,), jnp.int32),
            pltpu.VMEM((4 * C,), jnp.float32),
            pltpu.VMEM((4 * C,), jnp.float32),
            pltpu.VMEM((4 * C, F), jnp.float32),
            pltpu.VMEM((4 * C, F), jnp.float32),
            pltpu.SemaphoreType.DMA,
            pltpu.SemaphoreType.DMA,
        ],
        compiler_params=pltpu.CompilerParams(
            needs_layout_passes=False, use_tc_tiling_on_sc=False
        ),
    )(_body)
    return run2(px, py, cb2)


# speedup vs baseline: 4.9267x; 1.0210x over previous
import functools

import jax
import jax.numpy as jnp
from jax import lax
from jax.experimental import pallas as pl
from jax.experimental.pallas import tpu as pltpu
from jax.experimental.pallas import tpu_sc as plsc

R = 2 ** 18
F = 48
N = 1000000
C = 320
NCH = N // C
NW = 32
SEG = F // 16
NG = C // 16


def _body(px_hbm, py_hbm, cb_hbm, out_hbm,
          px_v, py_v, idx_a, idx_b, w_a, w_b, feat_a, feat_b,
          sem_a, sem_b):
    wid = lax.axis_index("s") * 2 + lax.axis_index("c")
    nch_w = (NCH - wid + NW - 1) // NW
    lanes = jnp.arange(16, dtype=jnp.int32)
    lim = jnp.float32(R - 1 - 1e-5)
    top = jnp.full((16,), R - 1, jnp.int32)

    def stage_idx(cid, idx_v, w_v):
        base = cid * C
        pltpu.sync_copy(px_hbm.at[pl.ds(base, C)], px_v)
        pltpu.sync_copy(py_hbm.at[pl.ds(base, C)], py_v)

        def idx_body(i, carry):
            sl = pl.ds(i * 16, 16)
            x = jnp.maximum(jnp.minimum(px_v[sl] * (R - 1.0), lim), 0.0)
            y = jnp.maximum(jnp.minimum(py_v[sl] * (R - 1.0), lim), 0.0)
            xi = x.astype(jnp.int32)
            yi = y.astype(jnp.int32)
            x2 = jnp.minimum(xi + 1, top)
            y2 = jnp.minimum(yi + 1, top)
            idx_v[sl] = xi * 2
            idx_v[pl.ds(C + i * 16, 16)] = x2 * 2
            idx_v[pl.ds(2 * C + i * 16, 16)] = yi * 2 + 1
            idx_v[pl.ds(3 * C + i * 16, 16)] = y2 * 2 + 1
            w_v[sl] = x2.astype(jnp.float32) - x
            w_v[pl.ds(C + i * 16, 16)] = x - xi.astype(jnp.float32)
            w_v[pl.ds(2 * C + i * 16, 16)] = y2.astype(jnp.float32) - y
            w_v[pl.ds(3 * C + i * 16, 16)] = y - yi.astype(jnp.float32)
            return carry

        lax.fori_loop(0, NG, idx_body, 0)

    def stage_combine_out(cid, w_v, feat_v):
        def comb_body(g, carry):
            w1g = w_v[pl.ds(g * 16, 16)]
            w2g = w_v[pl.ds(C + g * 16, 16)]
            w3g = w_v[pl.ds(2 * C + g * 16, 16)]
            w4g = w_v[pl.ds(3 * C + g * 16, 16)]
            for l in range(16):
                c = g * 16 + l
                w1 = w1g[l]
                w2 = w2g[l]
                w3 = w3g[l]
                w4 = w4g[l]
                for s in range(SEG):
                    sl = pl.ds(s * 16, 16)
                    a = feat_v[c, sl]
                    b = feat_v[C + c, sl]
                    cc = feat_v[2 * C + c, sl]
                    d = feat_v[3 * C + c, sl]
                    feat_v[c, sl] = (w1 * a + w2 * b) + (w3 * cc + w4 * d)
            return carry

        lax.fori_loop(0, NG, comb_body, 0)
        pltpu.sync_copy(feat_v.at[pl.ds(0, C)],
                        out_hbm.at[pl.ds(cid * C, C)])

    def gather_start(idx_v, feat_v, sem):
        return pltpu.async_copy(cb_hbm.at[idx_v], feat_v, sem)

    stage_idx(wid, idx_a, w_a)
    gather_start(idx_a, feat_a, sem_a)

    npairs = (nch_w + 1) // 2

    def pair_body(j, carry):
        ka = 2 * j
        kb = 2 * j + 1
        cid_a = wid + ka * NW
        cid_b = wid + kb * NW

        @pl.when(kb < nch_w)
        def _():
            stage_idx(cid_b, idx_b, w_b)
            gather_start(idx_b, feat_b, sem_b)

        pltpu.make_async_copy(cb_hbm.at[idx_a], feat_a, sem_a).wait()
        stage_combine_out(cid_a, w_a, feat_a)

        @pl.when(kb < nch_w)
        def _():
            @pl.when(kb + 1 < nch_w)
            def _():
                stage_idx(cid_b + NW, idx_a, w_a)
                gather_start(idx_a, feat_a, sem_a)

            pltpu.make_async_copy(cb_hbm.at[idx_b], feat_b, sem_b).wait()
            stage_combine_out(cid_b, w_b, feat_b)

        return carry

    lax.fori_loop(0, npairs, pair_body, 0)


CC = 256
CPW = R // NW


def _tbody(cbt_hbm, out_hbm, in_a, in_b, out_a, out_b,
           sem_a, sem_b, osem_a, osem_b):
    wid = lax.axis_index("s") * 2 + lax.axis_index("c")
    col0 = wid * CPW
    nch = CPW // CC

    def fetch(k, buf, sem):
        return pltpu.async_copy(
            cbt_hbm.at[:, pl.ds(col0 + k * CC, CC)], buf, sem)

    lanes = jnp.arange(16, dtype=jnp.int32)
    rots = [jnp.bitwise_and(lanes + k, 15) for k in range(16)]
    outv = [r * 96 + lanes for r in rots]

    def transpose(in_v, out_v):
        def cb_body(cb, carry):
            c0 = cb * 16
            for f0 in range(0, 96, 16):
                fi = lanes + f0
                for kk in range(16):
                    v = plsc.load_gather(in_v, [fi, c0 + rots[kk]])
                    plsc.store_scatter(out_v, [(c0 * 96 + f0) + outv[kk]], v)
            return carry

        lax.fori_loop(0, CC // 16, cb_body, 0)

    def out_copy(k, out_v, osem):
        return pltpu.async_copy(
            out_v, out_hbm.at[pl.ds((col0 + k * CC) * 96, CC * 96)], osem)

    def out_wait(k, out_v, osem):
        pltpu.make_async_copy(
            out_v, out_hbm.at[pl.ds((col0 + k * CC) * 96, CC * 96)],
            osem).wait()

    fetch(0, in_a, sem_a)

    def pair_body(j, carry):
        ka = 2 * j
        kb = 2 * j + 1
        pltpu.make_async_copy(
            cbt_hbm.at[:, pl.ds(col0 + ka * CC, CC)], in_a, sem_a).wait()
        fetch(kb, in_b, sem_b)

        @pl.when(j > 0)
        def _():
            out_wait(ka - 2, out_a, osem_a)

        transpose(in_a, out_a)
        out_copy(ka, out_a, osem_a)

        @pl.when(kb + 1 < nch)
        def _():
            fetch(kb + 1, in_a, sem_a)

        pltpu.make_async_copy(
            cbt_hbm.at[:, pl.ds(col0 + kb * CC, CC)], in_b, sem_b).wait()

        @pl.when(j > 0)
        def _():
            out_wait(kb - 2, out_b, osem_b)

        transpose(in_b, out_b)
        out_copy(kb, out_b, osem_b)
        return carry

    lax.fori_loop(0, nch // 2, pair_body, 0)
    out_wait(nch - 2, out_a, osem_a)
    out_wait(nch - 1, out_b, osem_b)


def _transpose_table(cbt):
    run = functools.partial(
        pl.kernel,
        mesh=plsc.VectorSubcoreMesh(core_axis_name="c", subcore_axis_name="s"),
        out_type=jax.ShapeDtypeStruct((2 * R * F,), jnp.float32),
        scratch_types=[
            pltpu.VMEM((96, CC), jnp.float32),
            pltpu.VMEM((96, CC), jnp.float32),
            pltpu.VMEM((96 * CC,), jnp.float32),
            pltpu.VMEM((96 * CC,), jnp.float32),
            pltpu.SemaphoreType.DMA,
            pltpu.SemaphoreType.DMA,
            pltpu.SemaphoreType.DMA,
            pltpu.SemaphoreType.DMA,
        ],
        compiler_params=pltpu.CompilerParams(
            needs_layout_passes=False, use_tc_tiling_on_sc=True
        ),
    )(_tbody)
    return run(cbt)


@jax.jit
def kernel(pts, codebook_0):
    cb2 = _transpose_table(codebook_0.T).reshape(2 * R, F)
    px = pts[:, 0]
    py = pts[:, 1]
    run2 = functools.partial(
        pl.kernel,
        mesh=plsc.VectorSubcoreMesh(core_axis_name="c", subcore_axis_name="s"),
        out_type=jax.ShapeDtypeStruct((N, F), jnp.float32),
        scratch_types=[
            pltpu.VMEM((C,), jnp.float32),
            pltpu.VMEM((C,), jnp.float32),
            pltpu.VMEM((4 * C,), jnp.int32),
            pltpu.VMEM((4 * C,), jnp.int32),
            pltpu.VMEM((4 * C,), jnp.float32),
            pltpu.VMEM((4 * C,), jnp.float32),
            pltpu.VMEM((4 * C, F), jnp.float32),
            pltpu.VMEM((4 * C, F), jnp.float32),
            pltpu.SemaphoreType.DMA,
            pltpu.SemaphoreType.DMA,
        ],
        compiler_params=pltpu.CompilerParams(
            needs_layout_passes=False, use_tc_tiling_on_sc=False
        ),
    )(_body)
    return run2(px, py, cb2)
